# baseline (speedup 1.0000x reference)
import functools
import math

import jax
import jax.numpy as jnp
import numpy as np
from jax.experimental import pallas as pl
from jax.experimental.pallas import tpu as pltpu

_VMEM_LIMIT_BYTES = 48 * 1024 * 1024


def _interp_matrix(out_size: int, in_size: int) -> np.ndarray:
    if out_size == 1 or in_size == 1:
        src = np.zeros((out_size,), dtype=np.float64)
    else:
        src = np.arange(out_size, dtype=np.float64) * (in_size - 1) / (out_size - 1)
    i0 = np.clip(np.floor(src).astype(np.int64), 0, in_size - 1)
    i1 = np.clip(i0 + 1, 0, in_size - 1)
    frac = src - i0
    a = np.zeros((out_size, in_size), dtype=np.float64)
    a[np.arange(out_size), i0] += 1.0 - frac
    a[np.arange(out_size), i1] += frac
    return a


def _kron_interp_matrix(h_out, h_in, w_out, w_in) -> np.ndarray:
    ah = _interp_matrix(h_out, h_in)
    aw = _interp_matrix(w_out, w_in)
    m = np.einsum("Hh,Ww->hwHW", ah, aw)
    return m.reshape(h_in * w_in, h_out * w_out)


def _largest_divisor_leq(n: int, cap: int) -> int:
    cap = max(1, min(n, cap))
    for d in range(cap, 0, -1):
        if n % d == 0:
            return d
    return 1


def _fused_kernel(x_ref, skip_ref, m_ref, o_ref, *, jx):
    j = pl.program_id(1)

    @pl.when(j < jx)
    def _upsample():
        _, tb, h_in, w_in = x_ref.shape
        _, _, h_out, w_out = o_ref.shape
        xb = x_ref[0].reshape(tb, h_in * w_in).astype(jnp.bfloat16)
        acc = jnp.dot(xb, m_ref[...], preferred_element_type=jnp.float32)
        o_ref[...] = acc.reshape(1, tb, h_out, w_out)

    @pl.when(j >= jx)
    def _copy_skip():
        o_ref[...] = skip_ref[...]


def _upsample_only_kernel(x_ref, m_ref, o_ref):
    _, tb, h_in, w_in = x_ref.shape
    _, _, h_out, w_out = o_ref.shape
    xb = x_ref[0].reshape(tb, h_in * w_in).astype(jnp.bfloat16)
    acc = jnp.dot(xb, m_ref[...], preferred_element_type=jnp.float32)
    o_ref[...] = acc.reshape(1, tb, h_out, w_out)


def _upsample_align_corners(x, h_out, w_out):
    n, c, h_in, w_in = x.shape
    m = jnp.asarray(_kron_interp_matrix(h_out, h_in, w_out, w_in), dtype=jnp.bfloat16)
    tb = _largest_divisor_leq(c, 128)
    out = pl.pallas_call(
        _upsample_only_kernel,
        out_shape=jax.ShapeDtypeStruct((n, c, h_out, w_out), x.dtype),
        grid=(n, c // tb),
        in_specs=[
            pl.BlockSpec((1, tb, h_in, w_in), lambda i, j: (i, j, 0, 0)),
            pl.BlockSpec((h_in * w_in, h_out * w_out), lambda i, j: (0, 0)),
        ],
        out_specs=pl.BlockSpec((1, tb, h_out, w_out), lambda i, j: (i, j, 0, 0)),
        compiler_params=pltpu.CompilerParams(
            dimension_semantics=("parallel", "parallel"),
            vmem_limit_bytes=_VMEM_LIMIT_BYTES),
    )(x, m)
    return out


def kernel(x, skip):
    n, c_x, h_in, w_in = x.shape
    n2, c_s, h_out, w_out = skip.shape
    assert n == n2, (x.shape, skip.shape)

    tb = _largest_divisor_leq(math.gcd(c_x, c_s), 128)
    if tb < 2 or (h_in * w_in) % 8 != 0:
        up = _upsample_align_corners(x, h_out, w_out)
        return jnp.concatenate([up, skip], axis=1)

    jx = c_x // tb
    js = c_s // tb
    jt = jx + js
    c_total = c_x + c_s

    m = jnp.asarray(_kron_interp_matrix(h_out, h_in, w_out, w_in), dtype=jnp.bfloat16)

    def x_map(nn, j):
        return (nn, jnp.minimum(j, jx - 1), 0, 0)

    def skip_map(nn, j):
        return (nn, jnp.maximum(j - jx, 0), 0, 0)

    def m_map(nn, j):
        return (0, 0)

    def out_map(nn, j):
        return (nn, j, 0, 0)

    out = pl.pallas_call(
        functools.partial(_fused_kernel, jx=jx),
        out_shape=jax.ShapeDtypeStruct((n, c_total, h_out, w_out), x.dtype),
        grid=(n, jt),
        in_specs=[
            pl.BlockSpec((1, tb, h_in, w_in), x_map),
            pl.BlockSpec((1, tb, h_out, w_out), skip_map),
            pl.BlockSpec((h_in * w_in, h_out * w_out), m_map),
        ],
        out_specs=pl.BlockSpec((1, tb, h_out, w_out), out_map),
        compiler_params=pltpu.CompilerParams(
            dimension_semantics=("parallel", "parallel"),
            vmem_limit_bytes=_VMEM_LIMIT_BYTES),
    )(x, skip, m)

    return out


# speedup vs baseline: 8.0103x; 8.0103x over previous
import functools

import jax
import jax.numpy as jnp
import numpy as np
from jax.experimental import pallas as pl
from jax.experimental.pallas import tpu as pltpu

_VMEM_LIMIT_BYTES = 48 * 1024 * 1024


def _interp_taps(out_size: int, in_size: int):
    if out_size == 1 or in_size == 1:
        src = np.zeros((out_size,), dtype=np.float64)
    else:
        src = np.arange(out_size, dtype=np.float64) * (in_size - 1) / (out_size - 1)
    i0 = np.clip(np.floor(src).astype(np.int64), 0, in_size - 1)
    i1 = np.clip(i0 + 1, 0, in_size - 1)
    frac = src - i0
    return i0, i1, 1.0 - frac, frac


def _interp_matrix(out_size: int, in_size: int) -> np.ndarray:
    i0, i1, w0, w1 = _interp_taps(out_size, in_size)
    a = np.zeros((out_size, in_size), dtype=np.float64)
    a[np.arange(out_size), i0] += w0
    a[np.arange(out_size), i1] += w1
    return a.astype(np.float32)


def _fused_nhwc_kernel(x_ref, s_ref, aw_ref, o_ref, *, c_x, taps_h):
    xb = x_ref[0]
    h_in = xb.shape[0]

    awb = jnp.broadcast_to(aw_ref[...][None], (h_in,) + aw_ref.shape)
    t = jax.lax.dot_general(awb, xb, (((2,), (1,)), ((0,), (0,))),
                            preferred_element_type=jnp.float32)

    i0h, i1h, w0h, w1h = taps_h
    for h in range(len(i0h)):
        y = t[int(i0h[h])] * float(w0h[h]) + t[int(i1h[h])] * float(w1h[h])
        o_ref[0, h, :, :c_x] = y

    o_ref[0, :, :, c_x:] = s_ref[0]


def kernel(x, skip):
    n, c_x, h_in, w_in = x.shape
    n2, c_s, h_out, w_out = skip.shape
    assert n == n2, (x.shape, skip.shape)
    c_total = c_x + c_s

    x_t = jnp.transpose(x, (0, 2, 3, 1))
    skip_t = jnp.transpose(skip, (0, 2, 3, 1))

    a_w = jnp.asarray(_interp_matrix(w_out, w_in))
    taps_h = _interp_taps(h_out, h_in)

    body = functools.partial(_fused_nhwc_kernel, c_x=c_x, taps_h=taps_h)

    out_t = pl.pallas_call(
        body,
        out_shape=jax.ShapeDtypeStruct((n, h_out, w_out, c_total), x.dtype),
        grid=(n,),
        in_specs=[
            pl.BlockSpec((1, h_in, w_in, c_x), lambda i: (i, 0, 0, 0)),
            pl.BlockSpec((1, h_out, w_out, c_s), lambda i: (i, 0, 0, 0)),
            pl.BlockSpec((w_out, w_in), lambda i: (0, 0)),
        ],
        out_specs=pl.BlockSpec((1, h_out, w_out, c_total), lambda i: (i, 0, 0, 0)),
        compiler_params=pltpu.CompilerParams(
            dimension_semantics=("parallel",),
            vmem_limit_bytes=_VMEM_LIMIT_BYTES),
    )(x_t, skip_t, a_w)

    return jnp.transpose(out_t, (0, 3, 1, 2))
